# Initial kernel scaffold; baseline (speedup 1.0000x reference)
import jax
import jax.numpy as jnp
from jax.experimental import pallas as pl


def kernel(z, codebook):
    raise NotImplementedError("write your pallas kernel here")


# speedup vs baseline: 1.8340x; 1.8340x over previous
import jax
import jax.numpy as jnp
from jax.experimental import pallas as pl

_K = 1024
_D = 64
_B = 16
_H = 32
_W = 32
_BETA = 0.25
_N = _B * _H * _W
_RB = 2048
_STEPS = _N // _RB


def _vq_body(z_ref, cb_ref, zq_ref, idx_ref, loss_ref):
    zb = z_ref[...]
    cb = cb_ref[...]
    z2 = jnp.sum(zb * zb, axis=1, keepdims=True)
    c2 = jnp.sum(cb * cb, axis=1)
    s = jax.lax.dot_general(
        zb, cb, (((1,), (1,)), ((), ())),
        preferred_element_type=jnp.float32)
    d = (z2 + c2[None, :]) - 2.0 * s
    dmin = jnp.min(d, axis=1, keepdims=True)
    kio = jax.lax.broadcasted_iota(jnp.int32, d.shape, 1)
    idx = jnp.min(jnp.where(d == dmin, kio, _K), axis=1)
    oh = (kio == idx[:, None]).astype(jnp.float32)
    zq = jax.lax.dot_general(
        oh, cb, (((1,), (0,)), ((), ())),
        preferred_element_type=jnp.float32)
    zq_ref[...] = zb + (zq - zb)
    idx_ref[...] = idx.reshape(1, 1, _RB)
    part = jnp.sum((zq - zb) ** 2).reshape(1, 1)

    @pl.when(pl.program_id(0) == 0)
    def _init():
        loss_ref[...] = jnp.zeros((1, 1), jnp.float32)

    loss_ref[...] += part

    @pl.when(pl.program_id(0) == _STEPS - 1)
    def _finish():
        loss_ref[...] = loss_ref[...] * ((1.0 + _BETA) / float(_N * _D))


def kernel(z, codebook):
    Bz, Dz, Hz, Wz = z.shape
    z_flat = jnp.transpose(z, (0, 2, 3, 1)).reshape(-1, Dz)
    zq_flat, idx3, loss11 = pl.pallas_call(
        _vq_body,
        grid=(_STEPS,),
        in_specs=[
            pl.BlockSpec((_RB, _D), lambda i: (i, 0)),
            pl.BlockSpec((_K, _D), lambda i: (0, 0)),
        ],
        out_specs=[
            pl.BlockSpec((_RB, _D), lambda i: (i, 0)),
            pl.BlockSpec((1, 1, _RB), lambda i: (i, 0, 0)),
            pl.BlockSpec((1, 1), lambda i: (0, 0)),
        ],
        out_shape=[
            jax.ShapeDtypeStruct((_N, _D), jnp.float32),
            jax.ShapeDtypeStruct((_STEPS, 1, _RB), jnp.int32),
            jax.ShapeDtypeStruct((1, 1), jnp.float32),
        ],
    )(z_flat, codebook)
    z_q_st = jnp.transpose(zq_flat.reshape(Bz, Hz, Wz, Dz), (0, 3, 1, 2))
    loss = loss11.reshape(())
    indices = idx3.reshape(Bz, Hz * Wz)
    return (z_q_st, loss, indices)


# speedup vs baseline: 2.0236x; 1.1034x over previous
import jax
import jax.numpy as jnp
from jax.experimental import pallas as pl

_K = 1024
_D = 64
_B = 16
_H = 32
_W = 32
_BETA = 0.25
_HW = _H * _W
_N = _B * _HW


def _vq_body(z_ref, cb_ref, zq_ref, idx_ref, loss_ref):
    zb = z_ref[0]
    cb = cb_ref[...]
    z2 = jnp.sum(zb * zb, axis=0, keepdims=True)
    c2 = jnp.sum(cb * cb, axis=1, keepdims=True)
    s2 = jax.lax.dot_general(
        cb, zb + zb, (((1,), (0,)), ((), ())),
        preferred_element_type=jnp.float32)
    d = (z2 + c2) - s2
    dmin = jnp.min(d, axis=0, keepdims=True)
    kio = jax.lax.broadcasted_iota(jnp.int32, d.shape, 0).astype(jnp.float32)
    idxf = jnp.min(jnp.where(d == dmin, kio, float(_K)), axis=0, keepdims=True)
    oh = (kio == idxf).astype(jnp.float32)
    zq = jax.lax.dot_general(
        cb, oh, (((0,), (0,)), ((), ())),
        preferred_element_type=jnp.float32)
    zq_ref[0] = zb + (zq - zb)
    idx_ref[...] = idxf.astype(jnp.int32).reshape(1, 1, _HW)
    part = jnp.sum((zq - zb) ** 2).reshape(1, 1)

    @pl.when(pl.program_id(0) == 0)
    def _init():
        loss_ref[...] = jnp.zeros((1, 1), jnp.float32)

    loss_ref[...] += part

    @pl.when(pl.program_id(0) == _B - 1)
    def _finish():
        loss_ref[...] = loss_ref[...] * ((1.0 + _BETA) / float(_N * _D))


def kernel(z, codebook):
    Bz, Dz, Hz, Wz = z.shape
    z3 = z.reshape(Bz, Dz, Hz * Wz)
    zq3, idx3, loss11 = pl.pallas_call(
        _vq_body,
        grid=(_B,),
        in_specs=[
            pl.BlockSpec((1, _D, _HW), lambda i: (i, 0, 0)),
            pl.BlockSpec((_K, _D), lambda i: (0, 0)),
        ],
        out_specs=[
            pl.BlockSpec((1, _D, _HW), lambda i: (i, 0, 0)),
            pl.BlockSpec((1, 1, _HW), lambda i: (i, 0, 0)),
            pl.BlockSpec((1, 1), lambda i: (0, 0)),
        ],
        out_shape=[
            jax.ShapeDtypeStruct((_B, _D, _HW), jnp.float32),
            jax.ShapeDtypeStruct((_B, 1, _HW), jnp.int32),
            jax.ShapeDtypeStruct((1, 1), jnp.float32),
        ],
    )(z3, codebook)
    z_q_st = zq3.reshape(Bz, Dz, Hz, Wz)
    loss = loss11.reshape(())
    indices = idx3.reshape(Bz, Hz * Wz)
    return (z_q_st, loss, indices)
